# baseline (speedup 1.0000x reference)
import numpy as np
import jax
import jax.numpy as jnp
from jax.experimental import pallas as pl
from jax.experimental.pallas import tpu as pltpu

_B = 16384
_S = 7
_C = 30
_CELLS = _S * _S
_F = _CELLS * _C
_GX = 7.0
_GY = 30.0

_BS = 512
_NCORE = 2
_NB = _B // _BS
_NBC = _NB // _NCORE

_PL = 128

_P_CH = (0, 1, 2, 3, 5, 6, 7, 8, 4, 9)
_L_CH = (0, 1, 2, 3, 4)


def _build_consts():
    import ml_dtypes
    bf16 = ml_dtypes.bfloat16
    gm = np.zeros((_F, 3 * _PL), np.float32)
    gp = np.zeros((_F, len(_P_CH) * _PL), np.float32)
    gl = np.zeros((_F, len(_L_CH) * _PL), np.float32)
    for j in range(_CELLS):
        base = j * _C
        for c in range(0, 4):
            gm[base + c, 0 * _PL + j] = 1.0
        for c in range(5, 9):
            gm[base + c, 1 * _PL + j] = 1.0
        for c in range(10, 30):
            gm[base + c, 2 * _PL + j] = 1.0
        for k, c in enumerate(_P_CH):
            gp[base + c, k * _PL + j] = 1.0
        for k, c in enumerate(_L_CH):
            gl[base + c, k * _PL + j] = 1.0
    wh = np.zeros((1, _F), np.float32)
    for j in range(_CELLS):
        for c in (2, 3, 7, 8):
            wh[0, j * _C + c] = 1.0
    return (gm.astype(bf16), gp.astype(bf16), gl.astype(bf16), wh)


_GM_NP, _GP_NP, _GL_NP, _WH_NP = _build_consts()


def _plane(a, k):
    return a[:, _PL * k:_PL * (k + 1)]


def _iou(x1, y1, x2, y2, gx1, gy1, gx2, gy2):
    ix1 = jnp.maximum(x1, gx1)
    iy1 = jnp.maximum(y1, gy1)
    ix2 = jnp.minimum(x2, gx2)
    iy2 = jnp.minimum(y2, gy2)
    iw = jnp.maximum(ix2 - ix1, 0.0)
    ih = jnp.maximum(iy2 - iy1, 0.0)
    inter = iw * ih
    a1 = (x2 - x1) * (y2 - y1)
    a2 = (gx2 - gx1) * (gy2 - gy1)
    union = a1 + a2 - inter
    pos = inter > 0
    return jnp.where(pos, inter / jnp.where(pos, union, 1.0), 0.0)


def _yolo_kernel(p_ref, l_ref, gm_ref, gp_ref, gl_ref, wh_ref, out_ref):
    P = p_ref[...]
    L = l_ref[...]
    wh = wh_ref[...]

    t = P - L
    s = jnp.sqrt(P) - jnp.sqrt(L)
    d = t + wh * (s - t)
    e = d * d

    esum = jnp.dot(e.astype(jnp.bfloat16), gm_ref[...],
                   preferred_element_type=jnp.float32)
    pp = jnp.dot(P.astype(jnp.bfloat16), gp_ref[...],
                 preferred_element_type=jnp.float32)
    lp = jnp.dot(L.astype(jnp.bfloat16), gl_ref[...],
                 preferred_element_type=jnp.float32)

    ji = jax.lax.broadcasted_iota(jnp.int32, (1, _PL), 1)
    mgx = (ji // _S).astype(jnp.float32) * (1.0 / _GX)
    ngy = (ji % _S).astype(jnp.float32) * (1.0 / _GY)

    p0, p1, p2, p3 = (_plane(pp, k) for k in range(4))
    p5, p6, p7, p8 = (_plane(pp, k) for k in range(4, 8))
    p4 = _plane(pp, 8)
    p9 = _plane(pp, 9)
    l0, l1, l2, l3, l4 = (_plane(lp, k) for k in range(5))

    def boxes(a, b, w, h):
        cx = a * (1.0 / _GX) + mgx
        cy = b * (1.0 / _GY) + ngy
        return cx - 0.5 * w, cy - 0.5 * h, cx + 0.5 * w, cy + 0.5 * h

    b1 = boxes(p0, p1, p2, p3)
    b2 = boxes(p5, p6, p7, p8)
    gb = boxes(l0, l1, l2, l3)

    iou1 = _iou(*b1, *gb)
    iou2 = _iou(*b2, *gb)
    sel1 = iou1 >= iou2
    obj = l4 > 0.5

    coor = 5.0 * jnp.where(sel1, _plane(esum, 0), _plane(esum, 1))
    class_s = _plane(esum, 2)
    d1 = (p4 - iou1) ** 2
    d2 = (p9 - iou2) ** 2
    obj_confi = jnp.where(sel1, d1, d2)
    noobj_at_obj = 0.5 * jnp.where(sel1, d2, d1)
    noobj = 0.5 * (p4 * p4 + p9 * p9)

    per_cell = jnp.where(obj, coor + obj_confi + noobj_at_obj + class_s, noobj)
    per_cell = jnp.where(ji < _CELLS, per_cell, 0.0)

    out_ref[...] = jnp.sum(per_cell, axis=0, keepdims=True).reshape(1, 1, _PL)


def kernel(pred, labels):
    pf = pred.reshape(_B, _F)
    lf = labels.reshape(_B, _F)
    gm = jnp.asarray(_GM_NP)
    gp = jnp.asarray(_GP_NP)
    gl = jnp.asarray(_GL_NP)
    wh = jnp.asarray(_WH_NP)

    row_spec = pl.BlockSpec((_BS, _F), lambda i: (i, 0))

    def const_spec(shape):
        return pl.BlockSpec(shape, lambda i: tuple(0 for _ in shape))

    out = pl.pallas_call(
        _yolo_kernel,
        out_shape=jax.ShapeDtypeStruct((_NB, 1, _PL), jnp.float32),
        grid=(_NB,),
        in_specs=[
            row_spec,
            row_spec,
            const_spec(_GM_NP.shape),
            const_spec(_GP_NP.shape),
            const_spec(_GL_NP.shape),
            const_spec(_WH_NP.shape),
        ],
        out_specs=pl.BlockSpec((1, 1, _PL), lambda i: (i, 0, 0)),
        compiler_params=pltpu.CompilerParams(
            dimension_semantics=("arbitrary",),
            vmem_limit_bytes=50 * 1024 * 1024,
        ),
        name="yolo_v1_loss",
    )(pf, lf, gm, gp, gl, wh)

    return jnp.sum(out) * (1.0 / _B)


# speedup vs baseline: 6.5108x; 6.5108x over previous
import jax
import jax.numpy as jnp
from jax.experimental import pallas as pl
from jax.experimental.pallas import tpu as pltpu

_B = 16384
_S = 7
_C = 30
_CELLS = _S * _S
_GX = 7.0
_GY = 30.0

_BL = 1024
_NSTEP = _B // _BL


def _iou(x1, y1, x2, y2, gx1, gy1, gx2, gy2):
    ix1 = jnp.maximum(x1, gx1)
    iy1 = jnp.maximum(y1, gy1)
    ix2 = jnp.minimum(x2, gx2)
    iy2 = jnp.minimum(y2, gy2)
    iw = jnp.maximum(ix2 - ix1, 0.0)
    ih = jnp.maximum(iy2 - iy1, 0.0)
    inter = iw * ih
    a1 = (x2 - x1) * (y2 - y1)
    a2 = (gx2 - gx1) * (gy2 - gy1)
    union = a1 + a2 - inter
    pos = inter > 0
    return jnp.where(pos, inter / jnp.where(pos, union, 1.0), 0.0)


def _yolo_kernel(p_ref, l_ref, out_ref):
    def pch(c):
        return p_ref[:, c, :]

    def lch(c):
        return l_ref[:, c, :]

    cls = None
    for c in range(10, 30):
        dd = pch(c) - lch(c)
        sq = dd * dd
        cls = sq if cls is None else cls + sq

    p0, p1, p2, p3, p4 = pch(0), pch(1), pch(2), pch(3), pch(4)
    p5, p6, p7, p8, p9 = pch(5), pch(6), pch(7), pch(8), pch(9)
    l0, l1, l2, l3, l4 = lch(0), lch(1), lch(2), lch(3), lch(4)

    coor1 = ((p0 - l0) ** 2 + (p1 - l1) ** 2
             + (jnp.sqrt(p2) - jnp.sqrt(l2)) ** 2
             + (jnp.sqrt(p3) - jnp.sqrt(l3)) ** 2)
    coor2 = ((p5 - lch(5)) ** 2 + (p6 - lch(6)) ** 2
             + (jnp.sqrt(p7) - jnp.sqrt(lch(7))) ** 2
             + (jnp.sqrt(p8) - jnp.sqrt(lch(8))) ** 2)

    def boxes(a, b, w, h):
        cx = a * (1.0 / _GX)
        cy = b * (1.0 / _GY)
        return cx - 0.5 * w, cy - 0.5 * h, cx + 0.5 * w, cy + 0.5 * h

    b1 = boxes(p0, p1, p2, p3)
    b2 = boxes(p5, p6, p7, p8)
    gb = boxes(l0, l1, l2, l3)

    iou1 = _iou(*b1, *gb)
    iou2 = _iou(*b2, *gb)
    sel1 = iou1 >= iou2
    obj = l4 == 1.0

    d1 = (p4 - iou1) ** 2
    d2 = (p9 - iou2) ** 2
    obj_branch = (5.0 * jnp.where(sel1, coor1, coor2)
                  + jnp.where(sel1, d1, d2)
                  + 0.5 * jnp.where(sel1, d2, d1)
                  + cls)
    noobj = 0.5 * (p4 * p4 + p9 * p9)
    per_cell = jnp.where(obj, obj_branch, noobj)

    out_ref[...] = jnp.sum(per_cell, axis=0, keepdims=True)


def kernel(pred, labels):
    pt = jnp.transpose(pred, (1, 2, 3, 0)).reshape(_CELLS, _C, _B)
    lt = jnp.transpose(labels, (1, 2, 3, 0)).reshape(_CELLS, _C, _B)

    in_spec = pl.BlockSpec((_CELLS, _C, _BL), lambda i: (0, 0, i))

    out = pl.pallas_call(
        _yolo_kernel,
        out_shape=jax.ShapeDtypeStruct((1, _B), jnp.float32),
        grid=(_NSTEP,),
        in_specs=[in_spec, in_spec],
        out_specs=pl.BlockSpec((1, _BL), lambda i: (0, i)),
        compiler_params=pltpu.CompilerParams(
            dimension_semantics=("arbitrary",),
            vmem_limit_bytes=50 * 1024 * 1024,
        ),
        name="yolo_v1_loss",
    )(pt, lt)

    return jnp.sum(out) * (1.0 / _B)


# speedup vs baseline: 7.2127x; 1.1078x over previous
import jax
import jax.numpy as jnp
from jax.experimental import pallas as pl
from jax.experimental.pallas import tpu as pltpu

_B = 16384
_S = 7
_C = 30
_CELLS = _S * _S
_GX = 7.0
_GY = 30.0

_BL = 1024
_NSTEP = _B // _BL
_CHUNK = 1024


def _iou_pair(pc0, pc1, pw, ph, lc0, lc1, lw, lh, a2):
    iw = jnp.minimum(jnp.minimum(
        0.5 * (pw + lw) - jnp.abs(pc0 - lc0) * (1.0 / _GX), pw), lw)
    ih = jnp.minimum(jnp.minimum(
        0.5 * (ph + lh) - jnp.abs(pc1 - lc1) * (1.0 / _GY), ph), lh)
    inter = jnp.maximum(iw, 0.0) * jnp.maximum(ih, 0.0)
    union = pw * ph + a2 - inter
    pos = inter > 0
    return jnp.where(pos, inter / jnp.where(pos, union, 1.0), 0.0)


def _yolo_kernel(p_ref, l_ref, out_ref):
    for h in range(_BL // _CHUNK):
        sl = slice(h * _CHUNK, (h + 1) * _CHUNK)

        def pch(c):
            return p_ref[:, c, sl]

        def lch(c):
            return l_ref[:, c, sl]

        dd = p_ref[:, 10:30, sl] - l_ref[:, 10:30, sl]
        cls = jnp.sum(dd * dd, axis=1)

        l0, l1, l2, l3 = lch(0), lch(1), lch(2), lch(3)
        a2 = l2 * l3

        p0, p1, p2, p3 = pch(0), pch(1), pch(2), pch(3)
        coor1 = ((p0 - l0) ** 2 + (p1 - l1) ** 2
                 + (jnp.sqrt(p2) - jnp.sqrt(l2)) ** 2
                 + (jnp.sqrt(p3) - jnp.sqrt(l3)) ** 2)
        iou1 = _iou_pair(p0, p1, p2, p3, l0, l1, l2, l3, a2)

        p5, p6, p7, p8 = pch(5), pch(6), pch(7), pch(8)
        coor2 = ((p5 - lch(5)) ** 2 + (p6 - lch(6)) ** 2
                 + (jnp.sqrt(p7) - jnp.sqrt(lch(7))) ** 2
                 + (jnp.sqrt(p8) - jnp.sqrt(lch(8))) ** 2)
        iou2 = _iou_pair(p5, p6, p7, p8, l0, l1, l2, l3, a2)

        sel1 = iou1 >= iou2

        p4, p9, l4 = pch(4), pch(9), lch(4)
        obj = l4 == 1.0
        d1 = (p4 - iou1) ** 2
        d2 = (p9 - iou2) ** 2
        obj_branch = (5.0 * jnp.where(sel1, coor1, coor2)
                      + jnp.where(sel1, d1, d2)
                      + 0.5 * jnp.where(sel1, d2, d1)
                      + cls)
        noobj = 0.5 * (p4 * p4 + p9 * p9)
        per_cell = jnp.where(obj, obj_branch, noobj)

        out_ref[0:1, sl] = jnp.sum(per_cell, axis=0, keepdims=True)


def kernel(pred, labels):
    pt = jnp.transpose(pred, (1, 2, 3, 0)).reshape(_CELLS, _C, _B)
    lt = jnp.transpose(labels, (1, 2, 3, 0)).reshape(_CELLS, _C, _B)

    in_spec = pl.BlockSpec((_CELLS, _C, _BL), lambda i: (0, 0, i))

    out = pl.pallas_call(
        _yolo_kernel,
        out_shape=jax.ShapeDtypeStruct((1, _B), jnp.float32),
        grid=(_NSTEP,),
        in_specs=[in_spec, in_spec],
        out_specs=pl.BlockSpec((1, _BL), lambda i: (0, i)),
        compiler_params=pltpu.CompilerParams(
            dimension_semantics=("arbitrary",),
            vmem_limit_bytes=50 * 1024 * 1024,
        ),
        name="yolo_v1_loss",
    )(pt, lt)

    return jnp.sum(out) * (1.0 / _B)
